# baseline (speedup 1.0000x reference)
import functools

import jax
import jax.numpy as jnp
from jax import lax
from jax.experimental import pallas as pl
from jax.experimental.pallas import tpu as pltpu
from jax.experimental.pallas import tpu_sc as plsc

D = 32
PITCH = 33
NT = 25
NC = 128
CPW = 4
NBLK = NT * CPW
NU = 8 * NBLK

_MESH = dict(core_axis_name="c", subcore_axis_name="s")


@functools.lru_cache(maxsize=None)
def _make_gather(V):
    @functools.partial(
        pl.kernel,
        mesh=plsc.VectorSubcoreMesh(**_MESH),
        out_type=jax.ShapeDtypeStruct((NT * 8 * NC, 128, D), jnp.float32),
        scratch_types=[
            pltpu.VMEM((16, 128), jnp.int32),
            pltpu.VMEM((16, 128, D), jnp.float32),
            pltpu.SemaphoreType.DMA,
            pltpu.SemaphoreType.DMA,
            pltpu.SemaphoreType.DMA,
        ],
        compiler_params=pltpu.CompilerParams(use_tc_tiling_on_sc=False),
    )
    def gather(idx_hbm, table_hbm, mid_hbm, idx_v, rows_v, g0, g1, wsem):
        w = lax.axis_index("s") * 2 + lax.axis_index("c")
        gsems = (g0, g1)

        def gather_copy(b, j):
            return pltpu.make_async_copy(
                table_hbm.at[idx_v.at[b * 8 + j]],
                rows_v.at[b * 8 + j],
                gsems[b],
            )

        def write_copy(blk, b, j):
            return pltpu.make_async_copy(
                rows_v.at[b * 8 + j],
                mid_hbm.at[(w * NBLK + blk) * 8 + j],
                wsem,
            )

        def fire(blk, b):
            T = blk // CPW
            C = w * CPW + blk % CPW

            @pl.when(blk >= 2)
            def _():
                for j in range(8):
                    write_copy(blk - 2, b, j).wait()

            pltpu.sync_copy(idx_hbm.at[T, C], idx_v.at[pl.ds(b * 8, 8)])
            for j in range(8):
                gather_copy(b, j).start()

        def process(blk, b):
            for j in range(8):
                gather_copy(b, j).wait()
                write_copy(blk, b, j).start()

        fire(0, 0)

        def body(tau, carry):
            i0 = 2 * tau

            @pl.when(i0 + 1 < NBLK)
            def _():
                fire(i0 + 1, 1)

            process(i0, 0)

            @pl.when(i0 + 2 < NBLK)
            def _():
                fire(i0 + 2, 0)

            process(i0 + 1, 1)
            return carry

        lax.fori_loop(0, NBLK // 2, body, 0)
        for j in range(8):
            write_copy(NBLK - 2, 0, j).wait()
            write_copy(NBLK - 1, 1, j).wait()

    return gather


@functools.lru_cache(maxsize=None)
def _make_transpose():
    @functools.partial(
        pl.kernel,
        mesh=plsc.VectorSubcoreMesh(**_MESH),
        out_type=jax.ShapeDtypeStruct((NT * 32, NC, 8, 128), jnp.float32),
        scratch_types=[
            pltpu.VMEM((8, 128, PITCH), jnp.float32),
            pltpu.VMEM((8, 128, PITCH), jnp.float32),
            pltpu.VMEM((32, 128), jnp.float32),
            pltpu.VMEM((32, 128), jnp.float32),
            pltpu.SemaphoreType.DMA,
            pltpu.SemaphoreType.DMA,
            pltpu.SemaphoreType.DMA,
            pltpu.SemaphoreType.DMA,
        ],
        compiler_params=pltpu.CompilerParams(
            use_tc_tiling_on_sc=False, needs_layout_passes=False
        ),
    )
    def transpose(mid_hbm, out_hbm, rows0, rows1, tiles0, tiles1,
                  r0, r1, w0, w1):
        w = lax.axis_index("s") * 2 + lax.axis_index("c")
        rows = (rows0, rows1)
        tiles = (tiles0, tiles1)
        rsems = (r0, r1)
        wsems = (w0, w1)
        iota = lax.iota(jnp.int32, 16)
        c_idx = [g * 16 + iota for g in range(8)]
        j_vec = [jnp.full((16,), j, dtype=jnp.int32) for j in range(8)]

        def read_copy(blk, b):
            return pltpu.make_async_copy(
                mid_hbm.at[pl.ds((w * NBLK + blk) * 8, 8)],
                rows[b].at[:, :, pl.ds(0, D)],
                rsems[b],
            )

        def tile_copy(blk, j, k):
            T = blk // CPW
            C = w * CPW + blk % CPW
            return pltpu.make_async_copy(
                tiles[j % 2].at[pl.ds(8 * k, 8)],
                out_hbm.at[32 * T + 4 * j + k, C],
                wsems[j % 2],
            )

        def process(blk, b):
            read_copy(blk, b).wait()
            for j in range(8):
                if j >= 2:
                    for k in range(4):
                        tile_copy(blk, j - 2, k).wait()
                else:

                    @pl.when(blk > 0)
                    def _(j=j):
                        for k in range(4):
                            tile_copy(blk - 1, j + 6, k).wait()

                src = rows[b]
                dst = tiles[j % 2]

                @plsc.parallel_loop(0, 32, unroll=8)
                def _t(d):
                    d_vec = jnp.full((16,), d, dtype=jnp.int32)
                    for g in range(8):
                        val = plsc.load_gather(
                            src, [j_vec[j], c_idx[g], d_vec]
                        )
                        dst[d, pl.ds(g * 16, 16)] = val

                for k in range(4):
                    tile_copy(blk, j, k).start()

        read_copy(0, 0).start()

        def body(tau, carry):
            i0 = 2 * tau

            @pl.when(i0 + 1 < NBLK)
            def _():
                read_copy(i0 + 1, 1).start()

            process(i0, 0)

            @pl.when(i0 + 2 < NBLK)
            def _():
                read_copy(i0 + 2, 0).start()

            process(i0 + 1, 1)
            return carry

        lax.fori_loop(0, NBLK // 2, body, 0)
        for k in range(4):
            tile_copy(NBLK - 1, 6, k).wait()
            tile_copy(NBLK - 1, 7, k).wait()

    return transpose


def kernel(token_ids, embedding):
    NB, NS = token_ids.shape
    idx4 = (
        token_ids.astype(jnp.int32)
        .reshape(NC, 128, NT, 8)
        .transpose(2, 0, 3, 1)
    )
    mid = _make_gather(embedding.shape[0])(idx4, embedding)
    out4 = _make_transpose()(mid)
    out = (
        out4.reshape(NS, 4, NC, 8, 128)
        .transpose(2, 4, 0, 1, 3)
        .reshape(NB, NS, D)
    )
    return out


# speedup vs baseline: 1.1377x; 1.1377x over previous
import functools

import jax
import jax.numpy as jnp
from jax import lax
from jax.experimental import pallas as pl
from jax.experimental.pallas import tpu as pltpu
from jax.experimental.pallas import tpu_sc as plsc

D = 32
PITCH = 49
NT = 25
NC = 128
CPW = 4
NBLK = NT * CPW
NU = 8 * NBLK

_MESH = dict(core_axis_name="c", subcore_axis_name="s")


@functools.lru_cache(maxsize=None)
def _make_gather(V):
    @functools.partial(
        pl.kernel,
        mesh=plsc.VectorSubcoreMesh(**_MESH),
        out_type=jax.ShapeDtypeStruct((NT * 8 * NC, 128, D), jnp.float32),
        scratch_types=[
            pltpu.VMEM((16, 128), jnp.int32),
            pltpu.VMEM((16, 128, D), jnp.float32),
            pltpu.SemaphoreType.DMA,
            pltpu.SemaphoreType.DMA,
            pltpu.SemaphoreType.DMA,
        ],
        compiler_params=pltpu.CompilerParams(use_tc_tiling_on_sc=False),
    )
    def gather(idx_hbm, table_hbm, mid_hbm, idx_v, rows_v, g0, g1, wsem):
        w = lax.axis_index("s") * 2 + lax.axis_index("c")
        gsems = (g0, g1)

        def gather_copy(b, j):
            return pltpu.make_async_copy(
                table_hbm.at[idx_v.at[b * 8 + j]],
                rows_v.at[b * 8 + j],
                gsems[b],
            )

        def write_copy(blk, b, j):
            return pltpu.make_async_copy(
                rows_v.at[b * 8 + j],
                mid_hbm.at[(w * NBLK + blk) * 8 + j],
                wsem,
            )

        def fire(blk, b):
            T = blk // CPW
            C = w * CPW + blk % CPW

            @pl.when(blk >= 2)
            def _():
                for j in range(8):
                    write_copy(blk - 2, b, j).wait()

            pltpu.sync_copy(idx_hbm.at[T, C], idx_v.at[pl.ds(b * 8, 8)])
            for j in range(8):
                gather_copy(b, j).start()

        def process(blk, b):
            for j in range(8):
                gather_copy(b, j).wait()
                write_copy(blk, b, j).start()

        fire(0, 0)

        def body(tau, carry):
            i0 = 2 * tau

            @pl.when(i0 + 1 < NBLK)
            def _():
                fire(i0 + 1, 1)

            process(i0, 0)

            @pl.when(i0 + 2 < NBLK)
            def _():
                fire(i0 + 2, 0)

            process(i0 + 1, 1)
            return carry

        lax.fori_loop(0, NBLK // 2, body, 0)
        for j in range(8):
            write_copy(NBLK - 2, 0, j).wait()
            write_copy(NBLK - 1, 1, j).wait()

    return gather


@functools.lru_cache(maxsize=None)
def _make_transpose():
    @functools.partial(
        pl.kernel,
        mesh=plsc.VectorSubcoreMesh(**_MESH),
        out_type=jax.ShapeDtypeStruct((NT * 32, NC, 8, 128), jnp.float32),
        scratch_types=[
            pltpu.VMEM((8, 128, PITCH), jnp.float32),
            pltpu.VMEM((8, 128, PITCH), jnp.float32),
            pltpu.VMEM((32, 128), jnp.float32),
            pltpu.VMEM((32, 128), jnp.float32),
            pltpu.SemaphoreType.DMA,
            pltpu.SemaphoreType.DMA,
            pltpu.SemaphoreType.DMA,
            pltpu.SemaphoreType.DMA,
        ],
        compiler_params=pltpu.CompilerParams(
            use_tc_tiling_on_sc=False, needs_layout_passes=False
        ),
    )
    def transpose(mid_hbm, out_hbm, rows0, rows1, tiles0, tiles1,
                  r0, r1, w0, w1):
        w = lax.axis_index("s") * 2 + lax.axis_index("c")
        rows = (rows0, rows1)
        tiles = (tiles0, tiles1)
        rsems = (r0, r1)
        wsems = (w0, w1)
        iota = lax.iota(jnp.int32, 16)
        c_idx = [g * 16 + iota for g in range(8)]
        j_vec = [jnp.full((16,), j, dtype=jnp.int32) for j in range(8)]

        def read_copy(blk, b):
            return pltpu.make_async_copy(
                mid_hbm.at[pl.ds((w * NBLK + blk) * 8, 8)],
                rows[b].at[:, :, pl.ds(0, D)],
                rsems[b],
            )

        def tile_copy(blk, j, k):
            T = blk // CPW
            C = w * CPW + blk % CPW
            return pltpu.make_async_copy(
                tiles[j % 2].at[pl.ds(8 * k, 8)],
                out_hbm.at[32 * T + 4 * j + k, C],
                wsems[j % 2],
            )

        def process(blk, b):
            read_copy(blk, b).wait()
            for j in range(8):
                if j >= 2:
                    for k in range(4):
                        tile_copy(blk, j - 2, k).wait()
                else:

                    @pl.when(blk > 0)
                    def _(j=j):
                        for k in range(4):
                            tile_copy(blk - 1, j + 6, k).wait()

                src = rows[b]
                dst = tiles[j % 2]

                @plsc.parallel_loop(0, 32, unroll=4)
                def _t(d):
                    d_vec = jnp.full((16,), d, dtype=jnp.int32)
                    for g in range(8):
                        val = plsc.load_gather(
                            src, [j_vec[j], c_idx[g], d_vec]
                        )
                        dst[d, pl.ds(g * 16, 16)] = val

                for k in range(4):
                    tile_copy(blk, j, k).start()

        read_copy(0, 0).start()

        def body(tau, carry):
            i0 = 2 * tau

            @pl.when(i0 + 1 < NBLK)
            def _():
                read_copy(i0 + 1, 1).start()

            process(i0, 0)

            @pl.when(i0 + 2 < NBLK)
            def _():
                read_copy(i0 + 2, 0).start()

            process(i0 + 1, 1)
            return carry

        lax.fori_loop(0, NBLK // 2, body, 0)
        for k in range(4):
            tile_copy(NBLK - 1, 6, k).wait()
            tile_copy(NBLK - 1, 7, k).wait()

    return transpose


def kernel(token_ids, embedding):
    NB, NS = token_ids.shape
    idx4 = (
        token_ids.astype(jnp.int32)
        .reshape(NC, 128, NT, 8)
        .transpose(2, 0, 3, 1)
    )
    mid = _make_gather(embedding.shape[0])(idx4, embedding)
    out4 = _make_transpose()(mid)
    out = (
        out4.reshape(NS, 4, NC, 8, 128)
        .transpose(2, 4, 0, 1, 3)
        .reshape(NB, NS, D)
    )
    return out
